# baseline (speedup 1.0000x reference)
import functools

import jax
import jax.numpy as jnp
from jax import lax
from jax.experimental import pallas as pl
from jax.experimental.pallas import tpu as pltpu
from jax.experimental.pallas import tpu_sc as plsc

N = 10000
DEG = 32
D = 128
K = 32
E = N * DEG

P = 5
NP = N // P
EP = NP * DEG

CHUNK = 80
NC, NS = 2, 16
NW = NC * NS
CH_PER_W = EP // (NW * CHUNK)
NBUF = 5
NGROUP = CH_PER_W // NBUF

BN = 125
BLK_P = NP // BN


def _poly_body(w_ref, c_ref, m_ref, prev_ref, o_ref):
    del prev_ref
    m3 = m_ref[...].astype(jnp.float32)
    m2 = m3.reshape(BN * DEG, D)
    proj = jnp.tanh(jnp.dot(m2, w_ref[...],
                            preferred_element_type=jnp.float32))
    logits = lax.dot_general(
        proj, c_ref[...],
        dimension_numbers=(((1,), (1,)), ((), ())),
        preferred_element_type=jnp.float32)
    l = logits.reshape(BN, DEG, K)
    mx = jnp.max(l, axis=1, keepdims=True)
    e = jnp.exp(l - mx)
    s = jnp.sum(e, axis=1, keepdims=True)
    w = e / s
    o_ref[...] = lax.dot_general(
        w, m3, dimension_numbers=(((1,), (1,)), ((0,), (0,))),
        preferred_element_type=jnp.float32)


def _sc_gather(src3d, xtab):
    mesh = plsc.VectorSubcoreMesh(core_axis_name="c", subcore_axis_name="s")

    @functools.partial(
        pl.kernel, mesh=mesh,
        out_type=jax.ShapeDtypeStruct((EP, D // 2), jnp.int32),
        compiler_params=pltpu.CompilerParams(use_tc_tiling_on_sc=False),
        scratch_types=[pltpu.VMEM((CH_PER_W, CHUNK), jnp.int32),
                       pltpu.VMEM((NBUF, CHUNK, D // 2), jnp.int32),
                       pltpu.SemaphoreType.DMA,
                       pltpu.SemaphoreType.DMA],
    )
    def k(src_hbm, x_hbm, mo_hbm, idx_all, xbuf, gsem, wsem):
        wid = lax.axis_index("s") * NC + lax.axis_index("c")
        base = wid * CH_PER_W
        pltpu.sync_copy(src_hbm.at[wid], idx_all)

        def group(g, carry):
            for b in range(NBUF):
                j = g * NBUF + b

                @pl.when(g > 0)
                def _():
                    pltpu.make_async_copy(
                        xbuf.at[b], mo_hbm.at[pl.ds(0, CHUNK)], wsem).wait()

                pltpu.async_copy(x_hbm.at[idx_all.at[j]], xbuf.at[b], gsem)
            for b in range(NBUF):
                j = g * NBUF + b
                pltpu.make_async_copy(
                    x_hbm.at[idx_all.at[j]], xbuf.at[b], gsem).wait()
                pltpu.async_copy(
                    xbuf.at[b], mo_hbm.at[pl.ds((base + j) * CHUNK, CHUNK)],
                    wsem)
            return carry

        lax.fori_loop(0, NGROUP, group, 0)
        for b in range(NBUF):
            pltpu.make_async_copy(
                xbuf.at[b], mo_hbm.at[pl.ds(0, CHUNK)], wsem).wait()

    return k(src3d, xtab)


def kernel(x, edge_index, W, context_codes):
    src = edge_index[0].astype(jnp.int32)
    src4d = src.reshape(P, NW, CH_PER_W, CHUNK)

    xb = lax.bitcast_convert_type(
        x.astype(jnp.bfloat16).reshape(N, D // 2, 2), jnp.int32)
    mails = [_sc_gather(src4d[p], xb) for p in range(P)]

    out = None
    for p in range(P):
        mail3 = lax.bitcast_convert_type(
            mails[p].reshape(NP, DEG, D // 2, 1), jnp.bfloat16).reshape(
                NP, DEG, D)
        in_specs = [pl.BlockSpec((D, D), lambda i: (0, 0)),
                    pl.BlockSpec((K, D), lambda i: (0, 0)),
                    pl.BlockSpec((BN, DEG, D), lambda i: (i, 0, 0))]
        operands = [W, context_codes, mail3]
        aliases = {}
        if p == 0:
            def body0(w_ref, c_ref, m_ref, o_ref):
                _poly_body(w_ref, c_ref, m_ref, None, o_ref)

            fn = body0
        else:
            in_specs.append(pl.BlockSpec(memory_space=pl.ANY))
            operands.append(out)
            aliases = {3: 0}
            fn = _poly_body
        out = pl.pallas_call(
            fn,
            grid=(BLK_P,),
            in_specs=in_specs,
            out_specs=pl.BlockSpec(
                (BN, K, D),
                functools.partial(lambda p_, i: (p_ * BLK_P + i, 0, 0), p)),
            out_shape=jax.ShapeDtypeStruct((N, K, D), jnp.float32),
            input_output_aliases=aliases,
        )(*operands)
    return out


# speedup vs baseline: 2.8960x; 2.8960x over previous
import functools

import jax
import jax.numpy as jnp
from jax import lax
from jax.experimental import pallas as pl
from jax.experimental.pallas import tpu as pltpu
from jax.experimental.pallas import tpu_sc as plsc

N = 10000
DEG = 32
D = 128
K = 32
E = N * DEG

P = 10
NP = N // P
EP = NP * DEG

CHUNK = 40
NC, NS = 2, 16
NW = NC * NS
CH_PER_W = EP // (NW * CHUNK)
NBUF = 5
NGROUP = CH_PER_W // NBUF

BN = 125
BLK_P = NP // BN


def _poly_body(w_ref, c_ref, m_ref, prev_ref, o_ref):
    del prev_ref
    m3 = m_ref[...]
    m2 = m3.reshape(BN * DEG, D)
    proj = jnp.tanh(jnp.dot(m2, w_ref[...],
                            preferred_element_type=jnp.float32))
    logits = lax.dot_general(
        proj, c_ref[...],
        dimension_numbers=(((1,), (1,)), ((), ())),
        preferred_element_type=jnp.float32)
    l = logits.reshape(BN, DEG, K)
    mx = jnp.max(l, axis=1, keepdims=True)
    e = jnp.exp(l - mx)
    s = jnp.sum(e, axis=1, keepdims=True)
    w = e / s
    o_ref[...] = lax.dot_general(
        w, m3, dimension_numbers=(((1,), (1,)), ((0,), (0,))),
        preferred_element_type=jnp.float32)


def _sc_gather(src3d, xtab):
    mesh = plsc.VectorSubcoreMesh(core_axis_name="c", subcore_axis_name="s")

    @functools.partial(
        pl.kernel, mesh=mesh,
        out_type=jax.ShapeDtypeStruct((EP, D), jnp.float32),
        scratch_types=[pltpu.VMEM((CH_PER_W, CHUNK), jnp.int32),
                       pltpu.VMEM((NBUF, CHUNK, D), jnp.float32),
                       pltpu.SemaphoreType.DMA,
                       pltpu.SemaphoreType.DMA],
    )
    def k(src_hbm, x_hbm, mo_hbm, idx_all, xbuf, gsem, wsem):
        wid = lax.axis_index("s") * NC + lax.axis_index("c")
        base = wid * CH_PER_W
        pltpu.sync_copy(src_hbm.at[wid], idx_all)

        def group(g, carry):
            for b in range(NBUF):
                j = g * NBUF + b

                @pl.when(g > 0)
                def _():
                    pltpu.make_async_copy(
                        xbuf.at[b], mo_hbm.at[pl.ds(0, CHUNK)], wsem).wait()

                pltpu.async_copy(x_hbm.at[idx_all.at[j]], xbuf.at[b], gsem)
            for b in range(NBUF):
                j = g * NBUF + b
                pltpu.make_async_copy(
                    x_hbm.at[idx_all.at[j]], xbuf.at[b], gsem).wait()
                pltpu.async_copy(
                    xbuf.at[b], mo_hbm.at[pl.ds((base + j) * CHUNK, CHUNK)],
                    wsem)
            return carry

        lax.fori_loop(0, NGROUP, group, 0)
        for b in range(NBUF):
            pltpu.make_async_copy(
                xbuf.at[b], mo_hbm.at[pl.ds(0, CHUNK)], wsem).wait()

    return k(src3d, xtab)


def kernel(x, edge_index, W, context_codes):
    src = edge_index[0].astype(jnp.int32)
    src4d = src.reshape(P, NW, CH_PER_W, CHUNK)

    mails = [_sc_gather(src4d[p], x) for p in range(P)]

    out = None
    for p in range(P):
        mail3 = mails[p].reshape(NP, DEG, D)
        in_specs = [pl.BlockSpec((D, D), lambda i: (0, 0)),
                    pl.BlockSpec((K, D), lambda i: (0, 0)),
                    pl.BlockSpec((BN, DEG, D), lambda i: (i, 0, 0))]
        operands = [W, context_codes, mail3]
        aliases = {}
        if p == 0:
            def body0(w_ref, c_ref, m_ref, o_ref):
                _poly_body(w_ref, c_ref, m_ref, None, o_ref)

            fn = body0
        else:
            in_specs.append(pl.BlockSpec(memory_space=pl.ANY))
            operands.append(out)
            aliases = {3: 0}
            fn = _poly_body
        out = pl.pallas_call(
            fn,
            grid=(BLK_P,),
            in_specs=in_specs,
            out_specs=pl.BlockSpec(
                (BN, K, D),
                functools.partial(lambda p_, i: (p_ * BLK_P + i, 0, 0), p)),
            out_shape=jax.ShapeDtypeStruct((N, K, D), jnp.float32),
            input_output_aliases=aliases,
        )(*operands)
    return out


# speedup vs baseline: 3.1457x; 1.0862x over previous
import functools

import jax
import jax.numpy as jnp
from jax import lax
from jax.experimental import pallas as pl
from jax.experimental.pallas import tpu as pltpu
from jax.experimental.pallas import tpu_sc as plsc

N = 10000
DEG = 32
D = 128
K = 32
E = N * DEG

P = 5
NP = N // P
EP = NP * DEG

CHUNK = 80
NC, NS = 2, 16
NW = NC * NS
CH_PER_W = EP // (NW * CHUNK)
NBUF = 5
NGROUP = CH_PER_W // NBUF

BN = 250
BLK_P = NP // BN


def _poly_body(w_ref, c_ref, m_ref, prev_ref, o_ref):
    del prev_ref
    m3 = m_ref[...]
    m2 = m3.reshape(BN * DEG, D)
    proj = jnp.tanh(jnp.dot(m2, w_ref[...],
                            preferred_element_type=jnp.float32))
    logits = lax.dot_general(
        proj, c_ref[...],
        dimension_numbers=(((1,), (1,)), ((), ())),
        preferred_element_type=jnp.float32)
    l = logits.reshape(BN, DEG, K)
    mx = jnp.max(l, axis=1, keepdims=True)
    e = jnp.exp(l - mx)
    s = jnp.sum(e, axis=1, keepdims=True)
    w = e / s
    o_ref[...] = lax.dot_general(
        w, m3, dimension_numbers=(((1,), (1,)), ((0,), (0,))),
        preferred_element_type=jnp.float32)


def _sc_gather(src3d, xtab):
    mesh = plsc.VectorSubcoreMesh(core_axis_name="c", subcore_axis_name="s")

    @functools.partial(
        pl.kernel, mesh=mesh,
        out_type=jax.ShapeDtypeStruct((EP, D), jnp.float32),
        scratch_types=[pltpu.VMEM((CH_PER_W, CHUNK), jnp.int32),
                       pltpu.VMEM((NBUF, CHUNK, D), jnp.float32),
                       pltpu.SemaphoreType.DMA,
                       pltpu.SemaphoreType.DMA],
    )
    def k(src_hbm, x_hbm, mo_hbm, idx_all, xbuf, gsem, wsem):
        wid = lax.axis_index("s") * NC + lax.axis_index("c")
        base = wid * CH_PER_W
        pltpu.sync_copy(src_hbm.at[wid], idx_all)

        def group(g, carry):
            for b in range(NBUF):
                j = g * NBUF + b

                @pl.when(g > 0)
                def _():
                    pltpu.make_async_copy(
                        xbuf.at[b], mo_hbm.at[pl.ds(0, CHUNK)], wsem).wait()

                pltpu.async_copy(x_hbm.at[idx_all.at[j]], xbuf.at[b], gsem)
            for b in range(NBUF):
                j = g * NBUF + b
                pltpu.make_async_copy(
                    x_hbm.at[idx_all.at[j]], xbuf.at[b], gsem).wait()
                pltpu.async_copy(
                    xbuf.at[b], mo_hbm.at[pl.ds((base + j) * CHUNK, CHUNK)],
                    wsem)
            return carry

        lax.fori_loop(0, NGROUP, group, 0)
        for b in range(NBUF):
            pltpu.make_async_copy(
                xbuf.at[b], mo_hbm.at[pl.ds(0, CHUNK)], wsem).wait()

    return k(src3d, xtab)


def kernel(x, edge_index, W, context_codes):
    src = edge_index[0].astype(jnp.int32)
    src4d = src.reshape(P, NW, CH_PER_W, CHUNK)

    mails = [_sc_gather(src4d[p], x) for p in range(P)]

    out = None
    for p in range(P):
        mail3 = mails[p].reshape(NP, DEG, D)
        in_specs = [pl.BlockSpec((D, D), lambda i: (0, 0)),
                    pl.BlockSpec((K, D), lambda i: (0, 0)),
                    pl.BlockSpec((BN, DEG, D), lambda i: (i, 0, 0))]
        operands = [W, context_codes, mail3]
        aliases = {}
        if p == 0:
            def body0(w_ref, c_ref, m_ref, o_ref):
                _poly_body(w_ref, c_ref, m_ref, None, o_ref)

            fn = body0
        else:
            in_specs.append(pl.BlockSpec(memory_space=pl.ANY))
            operands.append(out)
            aliases = {3: 0}
            fn = _poly_body
        out = pl.pallas_call(
            fn,
            grid=(BLK_P,),
            in_specs=in_specs,
            out_specs=pl.BlockSpec(
                (BN, K, D),
                functools.partial(lambda p_, i: (p_ * BLK_P + i, 0, 0), p)),
            out_shape=jax.ShapeDtypeStruct((N, K, D), jnp.float32),
            input_output_aliases=aliases,
        )(*operands)
    return out


# speedup vs baseline: 3.2560x; 1.0351x over previous
import functools

import jax
import jax.numpy as jnp
from jax import lax
from jax.experimental import pallas as pl
from jax.experimental.pallas import tpu as pltpu
from jax.experimental.pallas import tpu_sc as plsc

N = 10000
DEG = 32
D = 128
K = 32
E = N * DEG

P = 5
NP = N // P
EP = NP * DEG

CHUNK = 80
NC, NS = 2, 16
NW = NC * NS
CH_PER_W = EP // (NW * CHUNK)
NBUF = 5
NGROUP = CH_PER_W // NBUF

BN = 500
BLK_P = NP // BN


def _poly_body(w_ref, c_ref, m_ref, prev_ref, o_ref):
    del prev_ref
    m3 = m_ref[...]
    m2 = m3.reshape(BN * DEG, D)
    proj = jnp.tanh(jnp.dot(m2, w_ref[...],
                            preferred_element_type=jnp.float32))
    logits = lax.dot_general(
        proj, c_ref[...],
        dimension_numbers=(((1,), (1,)), ((), ())),
        preferred_element_type=jnp.float32)
    l = logits.reshape(BN, DEG, K)
    mx = jnp.max(l, axis=1, keepdims=True)
    e = jnp.exp(l - mx)
    s = jnp.sum(e, axis=1, keepdims=True)
    w = e / s
    o_ref[...] = lax.dot_general(
        w, m3, dimension_numbers=(((1,), (1,)), ((0,), (0,))),
        preferred_element_type=jnp.float32)


def _sc_gather(src3d, xtab):
    mesh = plsc.VectorSubcoreMesh(core_axis_name="c", subcore_axis_name="s")

    @functools.partial(
        pl.kernel, mesh=mesh,
        out_type=jax.ShapeDtypeStruct((EP, D), jnp.float32),
        scratch_types=[pltpu.VMEM((CH_PER_W, CHUNK), jnp.int32),
                       pltpu.VMEM((NBUF, CHUNK, D), jnp.float32),
                       pltpu.SemaphoreType.DMA,
                       pltpu.SemaphoreType.DMA],
    )
    def k(src_hbm, x_hbm, mo_hbm, idx_all, xbuf, gsem, wsem):
        wid = lax.axis_index("s") * NC + lax.axis_index("c")
        base = wid * CH_PER_W
        pltpu.sync_copy(src_hbm.at[wid], idx_all)

        def group(g, carry):
            for b in range(NBUF):
                j = g * NBUF + b

                @pl.when(g > 0)
                def _():
                    pltpu.make_async_copy(
                        xbuf.at[b], mo_hbm.at[pl.ds(0, CHUNK)], wsem).wait()

                pltpu.async_copy(x_hbm.at[idx_all.at[j]], xbuf.at[b], gsem)
            for b in range(NBUF):
                j = g * NBUF + b
                pltpu.make_async_copy(
                    x_hbm.at[idx_all.at[j]], xbuf.at[b], gsem).wait()
                pltpu.async_copy(
                    xbuf.at[b], mo_hbm.at[pl.ds((base + j) * CHUNK, CHUNK)],
                    wsem)
            return carry

        lax.fori_loop(0, NGROUP, group, 0)
        for b in range(NBUF):
            pltpu.make_async_copy(
                xbuf.at[b], mo_hbm.at[pl.ds(0, CHUNK)], wsem).wait()

    return k(src3d, xtab)


def kernel(x, edge_index, W, context_codes):
    src = edge_index[0].astype(jnp.int32)
    src4d = src.reshape(P, NW, CH_PER_W, CHUNK)

    mails = [_sc_gather(src4d[p], x) for p in range(P)]

    out = None
    for p in range(P):
        mail3 = mails[p].reshape(NP, DEG, D)
        in_specs = [pl.BlockSpec((D, D), lambda i: (0, 0)),
                    pl.BlockSpec((K, D), lambda i: (0, 0)),
                    pl.BlockSpec((BN, DEG, D), lambda i: (i, 0, 0))]
        operands = [W, context_codes, mail3]
        aliases = {}
        if p == 0:
            def body0(w_ref, c_ref, m_ref, o_ref):
                _poly_body(w_ref, c_ref, m_ref, None, o_ref)

            fn = body0
        else:
            in_specs.append(pl.BlockSpec(memory_space=pl.ANY))
            operands.append(out)
            aliases = {3: 0}
            fn = _poly_body
        out = pl.pallas_call(
            fn,
            grid=(BLK_P,),
            in_specs=in_specs,
            out_specs=pl.BlockSpec(
                (BN, K, D),
                functools.partial(lambda p_, i: (p_ * BLK_P + i, 0, 0), p)),
            out_shape=jax.ShapeDtypeStruct((N, K, D), jnp.float32),
            input_output_aliases=aliases,
        )(*operands)
    return out
